# baseline (speedup 1.0000x reference)
import functools

import jax
import jax.numpy as jnp
import numpy as np
from jax.experimental import pallas as pl

_TOPK_SENTINEL = 99
_NEG = -1000000000.0


def _bf16_dot(a, b):
    return jax.lax.dot_general(
        a.astype(jnp.bfloat16), b.astype(jnp.bfloat16),
        (((1,), (0,)), ((), ())), preferred_element_type=jnp.float32)


def _router_body(u_ref, pde_ref, did_ref, w1_ref, b1_ref, w2_ref, b2_ref,
                 emb_ref, trig_ref, out_ref, *, n_chan, fft_bins):
    u = u_ref[...]
    n = u.shape[1] * u.shape[2]
    blk = u.shape[0]

    s1 = jnp.sum(u, axis=(1, 2))
    s2 = jnp.sum(u * u, axis=(1, 2))
    mean = s1 / n
    var = (s2 - n * mean * mean) / (n - 1)
    std = jnp.sqrt(jnp.maximum(var, 0.0))
    amin = jnp.min(u, axis=(1, 2))
    amax = jnp.max(u, axis=(1, 2))

    trig = trig_ref[...]
    nq = u.shape[1]
    p = jnp.zeros((blk, trig.shape[2]), jnp.float32)
    for q in range(nq):
        p = p + jax.lax.dot_general(
            u[:, q, :], trig[q], (((1,), (0,)), ((), ())),
            preferred_element_type=jnp.float32,
            precision=jax.lax.Precision.HIGHEST)
    half = n_chan * fft_bins
    mag = jnp.zeros((blk, fft_bins), jnp.float32)
    for c in range(n_chan):
        re = p[:, c * fft_bins:(c + 1) * fft_bins]
        im = p[:, half + c * fft_bins: half + (c + 1) * fft_bins]
        mag = mag + jnp.sqrt(re * re + im * im)
    mag = mag * (1.0 / n_chan)

    did = did_ref[0, 0, :]
    n_ds = emb_ref.shape[0]
    onehot = (did[:, None] == jax.lax.broadcasted_iota(
        jnp.int32, (blk, n_ds), 1)).astype(jnp.float32)
    emb = jax.lax.dot_general(onehot, emb_ref[...], (((1,), (0,)), ((), ())),
                              preferred_element_type=jnp.float32,
                              precision=jax.lax.Precision.HIGHEST)

    x = jnp.concatenate([mean[:, None], std[:, None], amin[:, None],
                         amax[:, None], mag, pde_ref[...], emb], axis=1)
    h = jnp.maximum(_bf16_dot(x, w1_ref[...]) + b1_ref[...], 0.0)
    logits = _bf16_dot(h, w2_ref[...]) + b2_ref[...]

    n_exp = logits.shape[1]
    ids = jax.lax.broadcasted_iota(jnp.int32, (blk, n_exp), 1)
    m1 = jnp.max(logits, axis=1, keepdims=True)
    i1 = jnp.min(jnp.where(logits == m1, ids, _TOPK_SENTINEL),
                 axis=1, keepdims=True)
    rest = jnp.where(ids == i1, _NEG * 2.0, logits)
    m2 = jnp.max(rest, axis=1, keepdims=True)
    i2 = jnp.min(jnp.where(rest == m2, ids, _TOPK_SENTINEL),
                 axis=1, keepdims=True)
    keep = (ids == i1) | (ids == i2)
    masked = jnp.where(keep, logits, _NEG)

    e = jnp.exp(masked - m1)
    out_ref[...] = e / jnp.sum(e, axis=1, keepdims=True)


def _trig_matrix(S, C, fft_bins):
    i = np.arange(S * C)
    s = (i // C).astype(np.float64)
    k = np.arange(fft_bins, dtype=np.float64)
    ang = (2.0 * np.pi / S) * s[:, None] * k[None, :]
    chan_oh = (i[:, None] % C == np.arange(C)[None, :]).astype(np.float64)
    cos_blk = (chan_oh[:, :, None] * np.cos(ang)[:, None, :]).reshape(
        S * C, C * fft_bins)
    sin_blk = (chan_oh[:, :, None] * (-np.sin(ang))[:, None, :]).reshape(
        S * C, C * fft_bins)
    return np.concatenate([cos_blk, sin_blk], axis=1).astype(np.float32)


def kernel(u_state, pde_params, dataset_id, W1, b1, W2, b2, embed_table):
    B, S, C = u_state.shape
    fft_bins = 16
    n_exp = W2.shape[1]
    BLK = 256
    nb = B // BLK
    Q = S * C // 128

    trig = jnp.asarray(_trig_matrix(S, C, fft_bins).reshape(
        Q, 128, 2 * C * fft_bins))
    u3 = u_state.reshape(B, Q, 128)
    did3 = dataset_id.astype(jnp.int32).reshape(nb, 1, BLK)

    body = functools.partial(_router_body, n_chan=C, fft_bins=fft_bins)
    return pl.pallas_call(
        body,
        grid=(nb,),
        in_specs=[
            pl.BlockSpec((BLK, Q, 128), lambda i: (i, 0, 0)),
            pl.BlockSpec((BLK, pde_params.shape[1]), lambda i: (i, 0)),
            pl.BlockSpec((1, 1, BLK), lambda i: (i, 0, 0)),
            pl.BlockSpec(W1.shape, lambda i: (0, 0)),
            pl.BlockSpec((1, b1.shape[0]), lambda i: (0, 0)),
            pl.BlockSpec(W2.shape, lambda i: (0, 0)),
            pl.BlockSpec((1, b2.shape[0]), lambda i: (0, 0)),
            pl.BlockSpec(embed_table.shape, lambda i: (0, 0)),
            pl.BlockSpec((Q, 128, 2 * C * fft_bins), lambda i: (0, 0, 0)),
        ],
        out_specs=pl.BlockSpec((BLK, n_exp), lambda i: (i, 0)),
        out_shape=jax.ShapeDtypeStruct((B, n_exp), jnp.float32),
    )(u3, pde_params, did3, W1, b1.reshape(1, -1), W2, b2.reshape(1, -1),
      embed_table, trig)


# speedup vs baseline: 1.1684x; 1.1684x over previous
import functools

import jax
import jax.numpy as jnp
import numpy as np
from jax.experimental import pallas as pl
from jax.experimental.pallas import tpu as pltpu

_TOPK_SENTINEL = 99
_NEG = -1000000000.0


def _bf16_dot(a, b):
    return jax.lax.dot_general(
        a.astype(jnp.bfloat16), b.astype(jnp.bfloat16),
        (((1,), (0,)), ((), ())), preferred_element_type=jnp.float32)


def _router_body(u_ref, pde_ref, did_ref, w1_ref, b1_ref, w2_ref, b2_ref,
                 emb_ref, trig_ref, out_ref, *, n_chan, fft_bins):
    u = u_ref[...]
    n = u.shape[1]
    blk = u.shape[0]

    s1 = jnp.sum(u, axis=1)
    s2 = jnp.sum(u * u, axis=1)
    mean = s1 / n
    var = (s2 - n * mean * mean) / (n - 1)
    std = jnp.sqrt(jnp.maximum(var, 0.0))
    amin = jnp.min(u, axis=1)
    amax = jnp.max(u, axis=1)

    p = jax.lax.dot_general(u, trig_ref[...], (((1,), (0,)), ((), ())),
                            preferred_element_type=jnp.float32,
                            precision=jax.lax.Precision.HIGHEST)
    half = n_chan * fft_bins
    mag = jnp.zeros((blk, fft_bins), jnp.float32)
    for c in range(n_chan):
        re = p[:, c * fft_bins:(c + 1) * fft_bins]
        im = p[:, half + c * fft_bins: half + (c + 1) * fft_bins]
        mag = mag + jnp.sqrt(re * re + im * im)
    mag = mag * (1.0 / n_chan)

    did = did_ref[0, 0, :]
    n_ds = emb_ref.shape[0]
    onehot = (did[:, None] == jax.lax.broadcasted_iota(
        jnp.int32, (blk, n_ds), 1)).astype(jnp.float32)
    emb = jax.lax.dot_general(onehot, emb_ref[...], (((1,), (0,)), ((), ())),
                              preferred_element_type=jnp.float32,
                              precision=jax.lax.Precision.HIGHEST)

    x = jnp.concatenate([mean[:, None], std[:, None], amin[:, None],
                         amax[:, None], mag, pde_ref[...], emb], axis=1)
    h = jnp.maximum(_bf16_dot(x, w1_ref[...]) + b1_ref[...], 0.0)
    logits = _bf16_dot(h, w2_ref[...]) + b2_ref[...]

    n_exp = logits.shape[1]
    ids = jax.lax.broadcasted_iota(jnp.int32, (blk, n_exp), 1)
    m1 = jnp.max(logits, axis=1, keepdims=True)
    i1 = jnp.min(jnp.where(logits == m1, ids, _TOPK_SENTINEL),
                 axis=1, keepdims=True)
    rest = jnp.where(ids == i1, _NEG * 2.0, logits)
    m2 = jnp.max(rest, axis=1, keepdims=True)
    i2 = jnp.min(jnp.where(rest == m2, ids, _TOPK_SENTINEL),
                 axis=1, keepdims=True)
    keep = (ids == i1) | (ids == i2)
    masked = jnp.where(keep, logits, _NEG)

    e = jnp.exp(masked - m1)
    out_ref[...] = e / jnp.sum(e, axis=1, keepdims=True)


def _trig_matrix(S, C, fft_bins):
    i = np.arange(S * C)
    s = (i // C).astype(np.float64)
    k = np.arange(fft_bins, dtype=np.float64)
    ang = (2.0 * np.pi / S) * s[:, None] * k[None, :]
    chan_oh = (i[:, None] % C == np.arange(C)[None, :]).astype(np.float64)
    cos_blk = (chan_oh[:, :, None] * np.cos(ang)[:, None, :]).reshape(
        S * C, C * fft_bins)
    sin_blk = (chan_oh[:, :, None] * (-np.sin(ang))[:, None, :]).reshape(
        S * C, C * fft_bins)
    return np.concatenate([cos_blk, sin_blk], axis=1).astype(np.float32)


def kernel(u_state, pde_params, dataset_id, W1, b1, W2, b2, embed_table):
    B, S, C = u_state.shape
    fft_bins = 16
    n_exp = W2.shape[1]
    BLK = 256
    nb = B // BLK

    trig = jnp.asarray(_trig_matrix(S, C, fft_bins))
    u2 = u_state.reshape(B, S * C)
    did3 = dataset_id.astype(jnp.int32).reshape(nb, 1, BLK)

    body = functools.partial(_router_body, n_chan=C, fft_bins=fft_bins)
    return pl.pallas_call(
        body,
        grid=(nb,),
        in_specs=[
            pl.BlockSpec((BLK, S * C), lambda i: (i, 0)),
            pl.BlockSpec((BLK, pde_params.shape[1]), lambda i: (i, 0)),
            pl.BlockSpec((1, 1, BLK), lambda i: (i, 0, 0)),
            pl.BlockSpec(W1.shape, lambda i: (0, 0)),
            pl.BlockSpec((1, b1.shape[0]), lambda i: (0, 0)),
            pl.BlockSpec(W2.shape, lambda i: (0, 0)),
            pl.BlockSpec((1, b2.shape[0]), lambda i: (0, 0)),
            pl.BlockSpec(embed_table.shape, lambda i: (0, 0)),
            pl.BlockSpec((S * C, 2 * C * fft_bins), lambda i: (0, 0)),
        ],
        out_specs=pl.BlockSpec((BLK, n_exp), lambda i: (i, 0)),
        out_shape=jax.ShapeDtypeStruct((B, n_exp), jnp.float32),
        compiler_params=pltpu.CompilerParams(
            dimension_semantics=("parallel",)),
    )(u2, pde_params, did3, W1, b1.reshape(1, -1), W2, b2.reshape(1, -1),
      embed_table, trig)


# speedup vs baseline: 1.1760x; 1.0065x over previous
import functools

import jax
import jax.numpy as jnp
import numpy as np
from jax.experimental import pallas as pl
from jax.experimental.pallas import tpu as pltpu

_TOPK_SENTINEL = 99
_NEG = -1000000000.0


def _bf16_dot(a, b):
    return jax.lax.dot_general(
        a.astype(jnp.bfloat16), b.astype(jnp.bfloat16),
        (((1,), (0,)), ((), ())), preferred_element_type=jnp.float32)


def _router_body(u_ref, pde_ref, did_ref, w1_ref, b1_ref, w2_ref, b2_ref,
                 emb_ref, trig_ref, out_ref, *, n_chan, fft_bins):
    u = u_ref[...]
    n = u.shape[1]
    blk = u.shape[0]

    s1 = jnp.sum(u, axis=1)
    s2 = jnp.sum(u * u, axis=1)
    mean = s1 / n
    var = (s2 - n * mean * mean) / (n - 1)
    std = jnp.sqrt(jnp.maximum(var, 0.0))
    amin = jnp.min(u, axis=1)
    amax = jnp.max(u, axis=1)

    p = jax.lax.dot_general(u, trig_ref[...], (((1,), (0,)), ((), ())),
                            preferred_element_type=jnp.float32,
                            precision=jax.lax.Precision.HIGHEST)
    half = n_chan * fft_bins
    mag = jnp.zeros((blk, fft_bins), jnp.float32)
    for c in range(n_chan):
        re = p[:, c * fft_bins:(c + 1) * fft_bins]
        im = p[:, half + c * fft_bins: half + (c + 1) * fft_bins]
        mag = mag + jnp.sqrt(re * re + im * im)
    mag = mag * (1.0 / n_chan)

    did = did_ref[0, 0, :]
    n_ds = emb_ref.shape[0]
    onehot = (did[:, None] == jax.lax.broadcasted_iota(
        jnp.int32, (blk, n_ds), 1)).astype(jnp.float32)
    emb = jax.lax.dot_general(onehot, emb_ref[...], (((1,), (0,)), ((), ())),
                              preferred_element_type=jnp.float32,
                              precision=jax.lax.Precision.HIGHEST)

    x = jnp.concatenate([mean[:, None], std[:, None], amin[:, None],
                         amax[:, None], mag, pde_ref[...], emb], axis=1)
    h = jnp.maximum(_bf16_dot(x, w1_ref[...]) + b1_ref[...], 0.0)
    logits = _bf16_dot(h, w2_ref[...]) + b2_ref[...]

    n_exp = logits.shape[1]
    ids = jax.lax.broadcasted_iota(jnp.int32, (blk, n_exp), 1)
    m1 = jnp.max(logits, axis=1, keepdims=True)
    i1 = jnp.min(jnp.where(logits == m1, ids, _TOPK_SENTINEL),
                 axis=1, keepdims=True)
    rest = jnp.where(ids == i1, _NEG * 2.0, logits)
    m2 = jnp.max(rest, axis=1, keepdims=True)
    i2 = jnp.min(jnp.where(rest == m2, ids, _TOPK_SENTINEL),
                 axis=1, keepdims=True)
    keep = (ids == i1) | (ids == i2)
    masked = jnp.where(keep, logits, _NEG)

    e = jnp.exp(masked - m1)
    out_ref[...] = e / jnp.sum(e, axis=1, keepdims=True)


def _trig_matrix(S, C, fft_bins):
    i = np.arange(S * C)
    s = (i // C).astype(np.float64)
    k = np.arange(fft_bins, dtype=np.float64)
    ang = (2.0 * np.pi / S) * s[:, None] * k[None, :]
    chan_oh = (i[:, None] % C == np.arange(C)[None, :]).astype(np.float64)
    cos_blk = (chan_oh[:, :, None] * np.cos(ang)[:, None, :]).reshape(
        S * C, C * fft_bins)
    sin_blk = (chan_oh[:, :, None] * (-np.sin(ang))[:, None, :]).reshape(
        S * C, C * fft_bins)
    return np.concatenate([cos_blk, sin_blk], axis=1).astype(np.float32)


def kernel(u_state, pde_params, dataset_id, W1, b1, W2, b2, embed_table):
    B, S, C = u_state.shape
    fft_bins = 16
    n_exp = W2.shape[1]
    BLK = 512
    nb = B // BLK

    trig = jnp.asarray(_trig_matrix(S, C, fft_bins))
    u2 = u_state.reshape(B, S * C)
    did3 = dataset_id.astype(jnp.int32).reshape(nb, 1, BLK)

    body = functools.partial(_router_body, n_chan=C, fft_bins=fft_bins)
    return pl.pallas_call(
        body,
        grid=(nb,),
        in_specs=[
            pl.BlockSpec((BLK, S * C), lambda i: (i, 0)),
            pl.BlockSpec((BLK, pde_params.shape[1]), lambda i: (i, 0)),
            pl.BlockSpec((1, 1, BLK), lambda i: (i, 0, 0)),
            pl.BlockSpec(W1.shape, lambda i: (0, 0)),
            pl.BlockSpec((1, b1.shape[0]), lambda i: (0, 0)),
            pl.BlockSpec(W2.shape, lambda i: (0, 0)),
            pl.BlockSpec((1, b2.shape[0]), lambda i: (0, 0)),
            pl.BlockSpec(embed_table.shape, lambda i: (0, 0)),
            pl.BlockSpec((S * C, 2 * C * fft_bins), lambda i: (0, 0)),
        ],
        out_specs=pl.BlockSpec((BLK, n_exp), lambda i: (i, 0)),
        out_shape=jax.ShapeDtypeStruct((B, n_exp), jnp.float32),
        compiler_params=pltpu.CompilerParams(
            dimension_semantics=("parallel",)),
    )(u2, pde_params, did3, W1, b1.reshape(1, -1), W2, b2.reshape(1, -1),
      embed_table, trig)
